# Initial kernel scaffold; baseline (speedup 1.0000x reference)
import jax
import jax.numpy as jnp
from jax.experimental import pallas as pl


def kernel(edge_index, adj_values, features, W, b):
    raise NotImplementedError("write your pallas kernel here")


# speedup vs baseline: 4.0222x; 4.0222x over previous
import functools

import jax
import jax.numpy as jnp
from jax import lax
from jax.experimental import pallas as pl
from jax.experimental.pallas import tpu as pltpu
from jax.experimental.pallas import tpu_sc as plsc

_NC = 2
_NS = 16
_NW = _NC * _NS
_K = 80


def _sc_spmm(rows, cols, vals, features):
    E = vals.shape[0]
    N, D = features.shape
    assert E % (_NW * _K) == 0
    assert D % 16 == 0
    ew = E // _NW
    nch = ew // _K
    rb = 128
    rpt = ((N + _NS * rb - 1) // (_NS * rb)) * rb
    npad = _NS * rpt
    ncp = rpt // rb
    ng = _K // 16

    mesh = plsc.VectorSubcoreMesh(core_axis_name="c", subcore_axis_name="s")

    @functools.partial(
        pl.kernel,
        out_type=jax.ShapeDtypeStruct((_NC * npad, D), jnp.float32),
        mesh=mesh,
        compiler_params=pltpu.CompilerParams(needs_layout_passes=False),
        scratch_types=[
            pltpu.VMEM((_K,), jnp.int32),
            pltpu.VMEM((_K,), jnp.int32),
            pltpu.VMEM((_K,), jnp.float32),
            pltpu.VMEM((_K, D), jnp.float32),
            pltpu.VMEM((rb, D), jnp.float32),
            pltpu.VMEM_SHARED((npad, D), jnp.float32),
            pltpu.SemaphoreType.DMA,
        ],
    )
    def spmm(rows_hbm, cols_hbm, vals_hbm, feat_hbm, out_hbm,
             cidx_v, ridx_v, vals_v, gbuf, zbuf, acc, gsem):
        cid = lax.axis_index("c")
        sid = lax.axis_index("s")
        wid = sid * _NC + cid

        def zrow(r, carry):
            for cc in range(D // 16):
                zbuf[r, pl.ds(cc * 16, 16)] = jnp.zeros((16,), jnp.float32)
            return carry
        lax.fori_loop(0, rb, zrow, 0)
        for j in range(ncp):
            pltpu.sync_copy(zbuf, acc.at[pl.ds(sid * rpt + j * rb, rb)])
        plsc.subcore_barrier()

        def chunk_body(ch, carry):
            eb = pl.multiple_of(wid * ew + ch * _K, 8)
            pltpu.sync_copy(cols_hbm.at[pl.ds(eb, _K)], cidx_v)
            pltpu.sync_copy(rows_hbm.at[pl.ds(eb, _K)], ridx_v)
            pltpu.sync_copy(vals_hbm.at[pl.ds(eb, _K)], vals_v)
            pltpu.async_copy(feat_hbm.at[cidx_v], gbuf, gsem).wait()

            def edge_body(e, c2):
                vv = plsc.load_gather(vals_v, [jnp.full((16,), e, jnp.int32)])
                for cc in range(D // 16):
                    sl = pl.ds(cc * 16, 16)
                    gbuf[e, sl] = gbuf[e, sl] * vv
                return c2
            lax.fori_loop(0, _K, edge_body, 0)

            pltpu.sync_copy(gbuf, acc.at[ridx_v], add=True)
            return carry
        lax.fori_loop(0, nch, chunk_body, 0)
        plsc.subcore_barrier()

        for j in range(ncp):
            r0 = sid * rpt + j * rb
            pltpu.sync_copy(acc.at[pl.ds(r0, rb)], zbuf)
            pltpu.sync_copy(zbuf, out_hbm.at[pl.ds(cid * npad + r0, rb)])

    return spmm(rows, cols, vals, features)


def _linear(p0, p1, W, b2d):
    N, D = p0.shape
    DO = W.shape[0]
    blk = 1000

    def body(p0_ref, p1_ref, w_ref, b_ref, o_ref):
        x = p0_ref[...] + p1_ref[...]
        y = lax.dot_general(x, w_ref[...], (((1,), (1,)), ((), ())),
                            preferred_element_type=jnp.float32)
        o_ref[...] = y + b_ref[...]

    return pl.pallas_call(
        body,
        grid=(N // blk,),
        in_specs=[
            pl.BlockSpec((blk, D), lambda i: (i, 0)),
            pl.BlockSpec((blk, D), lambda i: (i, 0)),
            pl.BlockSpec((DO, D), lambda i: (0, 0)),
            pl.BlockSpec((1, DO), lambda i: (0, 0)),
        ],
        out_specs=pl.BlockSpec((blk, DO), lambda i: (i, 0)),
        out_shape=jax.ShapeDtypeStruct((N, DO), jnp.float32),
    )(p0, p1, W, b2d)


def kernel(edge_index, adj_values, features, W, b):
    N = features.shape[0]
    partials = _sc_spmm(edge_index[0], edge_index[1], adj_values, features)
    npad = partials.shape[0] // _NC
    return _linear(partials[:N], partials[npad:npad + N], W, b.reshape(1, -1))


# speedup vs baseline: 6.2164x; 1.5455x over previous
import functools

import jax
import jax.numpy as jnp
from jax import lax
from jax.experimental import pallas as pl
from jax.experimental.pallas import tpu as pltpu
from jax.experimental.pallas import tpu_sc as plsc

_NC = 2
_NS = 16
_NW = _NC * _NS
_K = 80


def _sc_spmm(rows, cols, vals, features):
    E = vals.shape[0]
    N, D = features.shape
    assert E % (_NW * _K) == 0
    assert D % 16 == 0
    ew = E // _NW
    nch = ew // _K
    assert nch % 2 == 1
    npairs = (nch - 1) // 2
    rb = 128
    rpt = ((N + _NS * rb - 1) // (_NS * rb)) * rb
    npad = _NS * rpt
    ncp = rpt // rb

    vals2 = vals.reshape(_NW, ew)

    mesh = plsc.VectorSubcoreMesh(core_axis_name="c", subcore_axis_name="s")

    @functools.partial(
        pl.kernel,
        out_type=jax.ShapeDtypeStruct((_NC * npad, D), jnp.float32),
        mesh=mesh,
        compiler_params=pltpu.CompilerParams(needs_layout_passes=False),
        scratch_types=[
            pltpu.VMEM((_K,), jnp.int32),
            pltpu.VMEM((_K,), jnp.int32),
            pltpu.VMEM((_K,), jnp.int32),
            pltpu.VMEM((_K,), jnp.int32),
            pltpu.VMEM((ew,), jnp.float32),
            pltpu.VMEM((2, _K, D), jnp.float32),
            pltpu.VMEM((rb, D), jnp.float32),
            pltpu.VMEM_SHARED((npad, D), jnp.float32),
            pltpu.SemaphoreType.DMA,
            pltpu.SemaphoreType.DMA,
            pltpu.SemaphoreType.DMA,
            pltpu.SemaphoreType.DMA,
        ],
    )
    def spmm(rows_hbm, cols_hbm, vals_hbm, feat_hbm, out_hbm,
             cidx0, cidx1, ridx0, ridx1, vals_v, gbuf, zbuf, acc,
             gsem0, gsem1, ssem0, ssem1):
        cid = lax.axis_index("c")
        sid = lax.axis_index("s")
        wid = sid * _NC + cid

        pltpu.sync_copy(vals_hbm.at[wid], vals_v)

        def zrow(r, carry):
            for cc in range(D // 16):
                zbuf[r, pl.ds(cc * 16, 16)] = jnp.zeros((16,), jnp.float32)
            return carry
        lax.fori_loop(0, rb, zrow, 0)
        for j in range(ncp):
            pltpu.sync_copy(zbuf, acc.at[pl.ds(sid * rpt + j * rb, rb)])
        plsc.subcore_barrier()

        bufs = [gbuf.at[0], gbuf.at[1]]
        cidxs = [cidx0, cidx1]
        ridxs = [ridx0, ridx1]
        gsems = [gsem0, gsem1]
        ssems = [ssem0, ssem1]

        def start_gather(ch, b):
            eb = pl.multiple_of(wid * ew + ch * _K, 8)
            pltpu.sync_copy(cols_hbm.at[pl.ds(eb, _K)], cidxs[b])
            pltpu.sync_copy(rows_hbm.at[pl.ds(eb, _K)], ridxs[b])
            pltpu.async_copy(feat_hbm.at[cidxs[b]], bufs[b], gsems[b])

        def wait_gather(b):
            pltpu.make_async_copy(feat_hbm.at[cidxs[b]], bufs[b],
                                  gsems[b]).wait()

        def scale(ch, b):
            base = ch * _K
            buf = bufs[b]

            def edge_body(e, c2):
                vv = plsc.load_gather(
                    vals_v, [jnp.full((16,), base + e, jnp.int32)])
                for cc in range(D // 16):
                    sl = pl.ds(cc * 16, 16)
                    buf[e, sl] = buf[e, sl] * vv
                return c2
            lax.fori_loop(0, _K, edge_body, 0)

        def start_scatter(ch, b):
            pltpu.sync_copy(bufs[b], acc.at[ridxs[b]], add=True)

        start_gather(0, 0)

        def pair_body(p, carry):
            ch0 = p * 2
            wait_gather(0)
            start_gather(ch0 + 1, 1)
            scale(ch0, 0)
            start_scatter(ch0, 0)
            wait_gather(1)
            start_gather(ch0 + 2, 0)
            scale(ch0 + 1, 1)
            start_scatter(ch0 + 1, 1)
            return carry
        lax.fori_loop(0, npairs, pair_body, 0)

        wait_gather(0)
        scale(nch - 1, 0)
        pltpu.sync_copy(bufs[0], acc.at[ridxs[0]], add=True)
        plsc.subcore_barrier()

        for j in range(ncp):
            r0 = sid * rpt + j * rb
            pltpu.sync_copy(acc.at[pl.ds(r0, rb)], zbuf)
            pltpu.sync_copy(zbuf, out_hbm.at[pl.ds(cid * npad + r0, rb)])

    return spmm(rows, cols, vals2, features)


def _linear(p0, p1, W, b2d):
    N, D = p0.shape
    DO = W.shape[0]
    blk = 1000

    def body(p0_ref, p1_ref, w_ref, b_ref, o_ref):
        x = p0_ref[...] + p1_ref[...]
        y = lax.dot_general(x, w_ref[...], (((1,), (1,)), ((), ())),
                            preferred_element_type=jnp.float32)
        o_ref[...] = y + b_ref[...]

    return pl.pallas_call(
        body,
        grid=(N // blk,),
        in_specs=[
            pl.BlockSpec((blk, D), lambda i: (i, 0)),
            pl.BlockSpec((blk, D), lambda i: (i, 0)),
            pl.BlockSpec((DO, D), lambda i: (0, 0)),
            pl.BlockSpec((1, DO), lambda i: (0, 0)),
        ],
        out_specs=pl.BlockSpec((blk, DO), lambda i: (i, 0)),
        out_shape=jax.ShapeDtypeStruct((N, DO), jnp.float32),
    )(p0, p1, W, b2d)


def kernel(edge_index, adj_values, features, W, b):
    N = features.shape[0]
    partials = _sc_spmm(edge_index[0], edge_index[1], adj_values, features)
    npad = partials.shape[0] // _NC
    return _linear(partials[:N], partials[npad:npad + N], W, b.reshape(1, -1))


# speedup vs baseline: 7.3428x; 1.1812x over previous
import functools

import jax
import jax.numpy as jnp
from jax import lax
from jax.experimental import pallas as pl
from jax.experimental.pallas import tpu as pltpu
from jax.experimental.pallas import tpu_sc as plsc

_NC = 2
_NS = 16
_NW = _NC * _NS
_K = 80


def _sc_spmm(rows, cols, vals, features):
    E = vals.shape[0]
    N, D = features.shape
    assert E % (_NW * _K) == 0
    assert D % 16 == 0
    ew = E // _NW
    nch = ew // _K
    assert nch % 2 == 1
    npairs = (nch - 1) // 2
    rb = 64
    rpt = ((N + _NS * rb - 1) // (_NS * rb)) * rb
    npad = _NS * rpt
    ncp = rpt // rb

    vals2 = vals.reshape(_NW, ew)

    mesh = plsc.VectorSubcoreMesh(core_axis_name="c", subcore_axis_name="s")

    @functools.partial(
        pl.kernel,
        out_type=jax.ShapeDtypeStruct((_NC * npad, D), jnp.float32),
        mesh=mesh,
        compiler_params=pltpu.CompilerParams(needs_layout_passes=False),
        scratch_types=[
            pltpu.VMEM((_K,), jnp.int32),
            pltpu.VMEM((_K,), jnp.int32),
            pltpu.VMEM((_K,), jnp.int32),
            pltpu.VMEM((_K,), jnp.int32),
            pltpu.VMEM((ew,), jnp.float32),
            pltpu.VMEM((2, _K, D), jnp.float32),
            pltpu.VMEM((rb, D), jnp.float32),
            pltpu.VMEM_SHARED((npad, D), jnp.float32),
            pltpu.SemaphoreType.DMA,
            pltpu.SemaphoreType.DMA,
            pltpu.SemaphoreType.DMA,
            pltpu.SemaphoreType.DMA,
        ],
    )
    def spmm(rows_hbm, cols_hbm, vals_hbm, feat_hbm, out_hbm,
             cidx0, cidx1, ridx0, ridx1, vals_v, gbuf, zbuf, acc,
             gsem0, gsem1, isem0, isem1):
        cid = lax.axis_index("c")
        sid = lax.axis_index("s")
        wid = sid * _NC + cid

        pltpu.sync_copy(vals_hbm.at[wid], vals_v)

        def zrow(r, carry):
            for cc in range(D // 16):
                zbuf[r, pl.ds(cc * 16, 16)] = jnp.zeros((16,), jnp.float32)
            return carry
        lax.fori_loop(0, rb, zrow, 0)
        for j in range(ncp):
            pltpu.sync_copy(zbuf, acc.at[pl.ds(sid * rpt + j * rb, rb)])
        plsc.subcore_barrier()

        bufs = [gbuf.at[0], gbuf.at[1]]
        cidxs = [cidx0, cidx1]
        ridxs = [ridx0, ridx1]
        gsems = [gsem0, gsem1]
        isems = [isem0, isem1]

        def start_idx(ch, b):
            eb = pl.multiple_of(wid * ew + ch * _K, 8)
            pltpu.async_copy(cols_hbm.at[pl.ds(eb, _K)], cidxs[b], isems[b])
            pltpu.async_copy(rows_hbm.at[pl.ds(eb, _K)], ridxs[b], isems[b])

        def wait_idx(ch, b):
            eb = pl.multiple_of(wid * ew + ch * _K, 8)
            pltpu.make_async_copy(cols_hbm.at[pl.ds(eb, _K)], cidxs[b],
                                  isems[b]).wait()
            pltpu.make_async_copy(rows_hbm.at[pl.ds(eb, _K)], ridxs[b],
                                  isems[b]).wait()

        def start_gather(ch, b):
            pltpu.async_copy(feat_hbm.at[cidxs[b]], bufs[b], gsems[b])

        def wait_gather(b):
            pltpu.make_async_copy(feat_hbm.at[cidxs[b]], bufs[b],
                                  gsems[b]).wait()

        def scale(ch, b):
            base = ch * _K
            buf = bufs[b]

            def edge_body(e, c2):
                vv = plsc.load_gather(
                    vals_v, [jnp.full((16,), base + e, jnp.int32)])
                for cc in range(D // 16):
                    sl = pl.ds(cc * 16, 16)
                    buf[e, sl] = buf[e, sl] * vv
                return c2
            lax.fori_loop(0, _K, edge_body, 0)

        def start_scatter(ch, b):
            pltpu.sync_copy(bufs[b], acc.at[ridxs[b]], add=True)

        start_idx(0, 0)
        wait_idx(0, 0)
        start_gather(0, 0)
        start_idx(1, 1)

        def pair_body(p, carry):
            ch0 = p * 2
            wait_gather(0)
            wait_idx(ch0 + 1, 1)
            start_gather(ch0 + 1, 1)
            scale(ch0, 0)
            start_scatter(ch0, 0)
            start_idx(ch0 + 2, 0)
            wait_gather(1)
            wait_idx(ch0 + 2, 0)
            start_gather(ch0 + 2, 0)
            scale(ch0 + 1, 1)
            start_scatter(ch0 + 1, 1)

            @pl.when(p < npairs - 1)
            def _():
                start_idx(ch0 + 3, 1)
            return carry
        lax.fori_loop(0, npairs, pair_body, 0)

        wait_gather(0)
        scale(nch - 1, 0)
        pltpu.sync_copy(bufs[0], acc.at[ridxs[0]], add=True)
        plsc.subcore_barrier()

        for j in range(ncp):
            r0 = sid * rpt + j * rb
            pltpu.sync_copy(acc.at[pl.ds(r0, rb)], zbuf)
            pltpu.sync_copy(zbuf, out_hbm.at[pl.ds(cid * npad + r0, rb)])

    return spmm(rows, cols, vals2, features)


def _linear(p0, p1, W, b2d):
    N, D = p0.shape
    DO = W.shape[0]
    blk = 1000

    def body(p0_ref, p1_ref, w_ref, b_ref, o_ref):
        x = p0_ref[...] + p1_ref[...]
        y = lax.dot_general(x, w_ref[...], (((1,), (1,)), ((), ())),
                            preferred_element_type=jnp.float32)
        o_ref[...] = y + b_ref[...]

    return pl.pallas_call(
        body,
        grid=(N // blk,),
        in_specs=[
            pl.BlockSpec((blk, D), lambda i: (i, 0)),
            pl.BlockSpec((blk, D), lambda i: (i, 0)),
            pl.BlockSpec((DO, D), lambda i: (0, 0)),
            pl.BlockSpec((1, DO), lambda i: (0, 0)),
        ],
        out_specs=pl.BlockSpec((blk, DO), lambda i: (i, 0)),
        out_shape=jax.ShapeDtypeStruct((N, DO), jnp.float32),
    )(p0, p1, W, b2d)


def kernel(edge_index, adj_values, features, W, b):
    N = features.shape[0]
    partials = _sc_spmm(edge_index[0], edge_index[1], adj_values, features)
    npad = partials.shape[0] // _NC
    return _linear(partials[:N], partials[npad:npad + N], W, b.reshape(1, -1))


# speedup vs baseline: 9.0616x; 1.2341x over previous
import functools

import jax
import jax.numpy as jnp
from jax import lax
from jax.experimental import pallas as pl
from jax.experimental.pallas import tpu as pltpu
from jax.experimental.pallas import tpu_sc as plsc

_NC = 2
_NS = 16
_NW = _NC * _NS
_K = 80


def _sc_spmm(rows, cols, vals, features):
    E = vals.shape[0]
    N, D = features.shape
    assert E % (_NW * _K) == 0
    assert D % 16 == 0
    ew = E // _NW
    nch = ew // _K
    ngroups = (nch - 5) // 6
    rem = nch - ngroups * 6
    assert rem == 5
    rb = 32
    rpt = ((N + _NS * rb - 1) // (_NS * rb)) * rb
    npad = _NS * rpt
    ncp = rpt // rb

    vals2 = vals.reshape(_NW, ew)

    mesh = plsc.VectorSubcoreMesh(core_axis_name="c", subcore_axis_name="s")

    @functools.partial(
        pl.kernel,
        out_type=jax.ShapeDtypeStruct((_NC * npad, D), jnp.float32),
        mesh=mesh,
        compiler_params=pltpu.CompilerParams(needs_layout_passes=False),
        scratch_types=(
            [pltpu.VMEM((_K,), jnp.int32)] * 12
            + [
                pltpu.VMEM((ew,), jnp.float32),
                pltpu.VMEM((3, _K, D), jnp.float32),
                pltpu.VMEM((rb, D), jnp.float32),
                pltpu.VMEM_SHARED((npad, D), jnp.float32),
            ]
            + [pltpu.SemaphoreType.DMA] * 9
        ),
    )
    def spmm(rows_hbm, cols_hbm, vals_hbm, feat_hbm, out_hbm,
             c0, c1, c2, c3, c4, c5, r0, r1, r2, r3, r4, r5,
             vals_v, gbuf, zbuf, acc,
             gs0, gs1, gs2, is0, is1, is2, is3, is4, is5):
        cid = lax.axis_index("c")
        sid = lax.axis_index("s")
        wid = sid * _NC + cid

        pltpu.sync_copy(vals_hbm.at[wid], vals_v)

        def zrow(r, carry):
            for cc in range(D // 16):
                zbuf[r, pl.ds(cc * 16, 16)] = jnp.zeros((16,), jnp.float32)
            return carry
        lax.fori_loop(0, rb, zrow, 0)
        for j in range(ncp):
            pltpu.sync_copy(zbuf, acc.at[pl.ds(sid * rpt + j * rb, rb)])
        plsc.subcore_barrier()

        bufs = [gbuf.at[0], gbuf.at[1], gbuf.at[2]]
        cidxs = [c0, c1, c2, c3, c4, c5]
        ridxs = [r0, r1, r2, r3, r4, r5]
        gsems = [gs0, gs1, gs2]
        isems = [is0, is1, is2, is3, is4, is5]

        def start_idx(ch, s):
            eb = pl.multiple_of(wid * ew + ch * _K, 8)
            pltpu.async_copy(cols_hbm.at[pl.ds(eb, _K)], cidxs[s], isems[s])
            pltpu.async_copy(rows_hbm.at[pl.ds(eb, _K)], ridxs[s], isems[s])

        def wait_idx(ch, s):
            eb = pl.multiple_of(wid * ew + ch * _K, 8)
            pltpu.make_async_copy(cols_hbm.at[pl.ds(eb, _K)], cidxs[s],
                                  isems[s]).wait()
            pltpu.make_async_copy(rows_hbm.at[pl.ds(eb, _K)], ridxs[s],
                                  isems[s]).wait()

        def start_gather(s, b):
            pltpu.async_copy(feat_hbm.at[cidxs[s]], bufs[b], gsems[b])

        def wait_gather(s, b):
            pltpu.make_async_copy(feat_hbm.at[cidxs[s]], bufs[b],
                                  gsems[b]).wait()

        def scale(ch, b):
            base = ch * _K
            buf = bufs[b]

            def edge_body(e, c2):
                vv = plsc.load_gather(
                    vals_v, [jnp.full((16,), base + e, jnp.int32)])
                for cc in range(D // 16):
                    sl = pl.ds(cc * 16, 16)
                    buf[e, sl] = buf[e, sl] * vv
                return c2
            lax.fori_loop(0, _K, edge_body, 0)

        def start_scatter(s, b):
            pltpu.sync_copy(bufs[b], acc.at[ridxs[s]], add=True)

        for s in range(6):
            start_idx(s, s)
        for b in range(3):
            wait_idx(b, b)
            start_gather(b, b)

        def group_body(g, carry):
            ch0 = g * 6
            for b in range(6):
                ch = ch0 + b
                b3 = b % 3
                s3 = (b + 3) % 6
                wait_gather(b, b3)
                scale(ch, b3)
                start_scatter(b, b3)

                @pl.when(ch + 6 < nch)
                def _():
                    start_idx(ch + 6, b)
                wait_idx(ch + 3, s3)
                start_gather(s3, b3)
            return carry
        lax.fori_loop(0, ngroups, group_body, 0)

        for b in range(rem):
            ch = ngroups * 6 + b
            b3 = b % 3
            s3 = (b + 3) % 6
            wait_gather(b, b3)
            scale(ch, b3)
            start_scatter(b, b3)
            if b < rem - 3:
                wait_idx(ch + 3, s3)
                start_gather(s3, b3)
        plsc.subcore_barrier()

        for j in range(ncp):
            rr = sid * rpt + j * rb
            pltpu.sync_copy(acc.at[pl.ds(rr, rb)], zbuf)
            pltpu.sync_copy(zbuf, out_hbm.at[pl.ds(cid * npad + rr, rb)])

    return spmm(rows, cols, vals2, features)


def _linear(p0, p1, W, b2d):
    N, D = p0.shape
    DO = W.shape[0]
    blk = 1000

    def body(p0_ref, p1_ref, w_ref, b_ref, o_ref):
        x = p0_ref[...] + p1_ref[...]
        y = lax.dot_general(x, w_ref[...], (((1,), (1,)), ((), ())),
                            preferred_element_type=jnp.float32)
        o_ref[...] = y + b_ref[...]

    return pl.pallas_call(
        body,
        grid=(N // blk,),
        in_specs=[
            pl.BlockSpec((blk, D), lambda i: (i, 0)),
            pl.BlockSpec((blk, D), lambda i: (i, 0)),
            pl.BlockSpec((DO, D), lambda i: (0, 0)),
            pl.BlockSpec((1, DO), lambda i: (0, 0)),
        ],
        out_specs=pl.BlockSpec((blk, DO), lambda i: (i, 0)),
        out_shape=jax.ShapeDtypeStruct((N, DO), jnp.float32),
    )(p0, p1, W, b2d)


def kernel(edge_index, adj_values, features, W, b):
    N = features.shape[0]
    partials = _sc_spmm(edge_index[0], edge_index[1], adj_values, features)
    npad = partials.shape[0] // _NC
    return _linear(partials[:N], partials[npad:npad + N], W, b.reshape(1, -1))


# speedup vs baseline: 11.1830x; 1.2341x over previous
import functools

import jax
import jax.numpy as jnp
from jax import lax
from jax.experimental import pallas as pl
from jax.experimental.pallas import tpu as pltpu
from jax.experimental.pallas import tpu_sc as plsc

_NC = 2
_NS = 16
_NW = _NC * _NS
_K = 80


def _sc_spmm(rows, cols, vals, features):
    E = vals.shape[0]
    N, D = features.shape
    assert E % (_NW * _K) == 0
    assert D % 16 == 0
    ew = E // _NW
    nch = ew // _K
    ngroups = (nch - 5) // 6
    rem = nch - ngroups * 6
    assert rem == 5
    rb = 32
    rpt = ((N + _NS * rb - 1) // (_NS * rb)) * rb
    npad = _NS * rpt
    ncp = rpt // rb

    vals2 = vals.reshape(_NW, ew)

    mesh = plsc.VectorSubcoreMesh(core_axis_name="c", subcore_axis_name="s")

    @functools.partial(
        pl.kernel,
        out_type=jax.ShapeDtypeStruct((_NC * npad, D), jnp.float32),
        mesh=mesh,
        compiler_params=pltpu.CompilerParams(needs_layout_passes=False),
        scratch_types=(
            [pltpu.VMEM((_K,), jnp.int32)] * 12
            + [
                pltpu.VMEM((ew,), jnp.float32),
                pltpu.VMEM((3, _K, D), jnp.float32),
                pltpu.VMEM((rb, D), jnp.float32),
                pltpu.VMEM_SHARED((npad, D), jnp.float32),
            ]
            + [pltpu.SemaphoreType.DMA] * 9
        ),
    )
    def spmm(rows_hbm, cols_hbm, vals_hbm, feat_hbm, out_hbm,
             c0, c1, c2, c3, c4, c5, r0, r1, r2, r3, r4, r5,
             vals_v, gbuf, zbuf, acc,
             gs0, gs1, gs2, is0, is1, is2, is3, is4, is5):
        cid = lax.axis_index("c")
        sid = lax.axis_index("s")
        wid = sid * _NC + cid

        pltpu.sync_copy(vals_hbm.at[wid], vals_v)

        def zrow(r, carry):
            for cc in range(D // 16):
                zbuf[r, pl.ds(cc * 16, 16)] = jnp.zeros((16,), jnp.float32)
            return carry
        lax.fori_loop(0, rb, zrow, 0)
        for j in range(ncp):
            pltpu.sync_copy(zbuf, acc.at[pl.ds(sid * rpt + j * rb, rb)])
        plsc.subcore_barrier()

        bufs = [gbuf.at[0], gbuf.at[1], gbuf.at[2]]
        cidxs = [c0, c1, c2, c3, c4, c5]
        ridxs = [r0, r1, r2, r3, r4, r5]
        gsems = [gs0, gs1, gs2]
        isems = [is0, is1, is2, is3, is4, is5]

        def start_idx(ch, s):
            eb = pl.multiple_of(wid * ew + ch * _K, 8)
            pltpu.async_copy(cols_hbm.at[pl.ds(eb, _K)], cidxs[s], isems[s])
            pltpu.async_copy(rows_hbm.at[pl.ds(eb, _K)], ridxs[s], isems[s])

        def wait_idx(ch, s):
            eb = pl.multiple_of(wid * ew + ch * _K, 8)
            pltpu.make_async_copy(cols_hbm.at[pl.ds(eb, _K)], cidxs[s],
                                  isems[s]).wait()
            pltpu.make_async_copy(rows_hbm.at[pl.ds(eb, _K)], ridxs[s],
                                  isems[s]).wait()

        def start_gather(s, b):
            pltpu.async_copy(feat_hbm.at[cidxs[s]], bufs[b], gsems[b])

        def wait_gather(s, b):
            pltpu.make_async_copy(feat_hbm.at[cidxs[s]], bufs[b],
                                  gsems[b]).wait()

        def scale(ch, b):
            base = ch * _K
            buf = bufs[b]
            unroll = 4

            def edge_body(i, c2):
                e0 = i * unroll
                vvs = [plsc.load_gather(
                    vals_v, [jnp.full((16,), base + e0 + u, jnp.int32)])
                    for u in range(unroll)]
                for cc in range(D // 16):
                    sl = pl.ds(cc * 16, 16)
                    for u in range(unroll):
                        buf[e0 + u, sl] = buf[e0 + u, sl] * vvs[u]
                return c2
            lax.fori_loop(0, _K // unroll, edge_body, 0)

        def start_scatter(s, b):
            pltpu.sync_copy(bufs[b], acc.at[ridxs[s]], add=True)

        for s in range(6):
            start_idx(s, s)
        for b in range(3):
            wait_idx(b, b)
            start_gather(b, b)

        def group_body(g, carry):
            ch0 = g * 6
            for b in range(6):
                ch = ch0 + b
                b3 = b % 3
                s3 = (b + 3) % 6
                wait_gather(b, b3)
                scale(ch, b3)
                start_scatter(b, b3)

                @pl.when(ch + 6 < nch)
                def _():
                    start_idx(ch + 6, b)
                wait_idx(ch + 3, s3)
                start_gather(s3, b3)
            return carry
        lax.fori_loop(0, ngroups, group_body, 0)

        for b in range(rem):
            ch = ngroups * 6 + b
            b3 = b % 3
            s3 = (b + 3) % 6
            wait_gather(b, b3)
            scale(ch, b3)
            start_scatter(b, b3)
            if b < rem - 3:
                wait_idx(ch + 3, s3)
                start_gather(s3, b3)
        plsc.subcore_barrier()

        for j in range(ncp):
            rr = sid * rpt + j * rb
            pltpu.sync_copy(acc.at[pl.ds(rr, rb)], zbuf)
            pltpu.sync_copy(zbuf, out_hbm.at[pl.ds(cid * npad + rr, rb)])

    return spmm(rows, cols, vals2, features)


def _linear(p0, p1, W, b2d):
    N, D = p0.shape
    DO = W.shape[0]
    blk = 1000

    def body(p0_ref, p1_ref, w_ref, b_ref, o_ref):
        x = p0_ref[...] + p1_ref[...]
        y = lax.dot_general(x, w_ref[...], (((1,), (1,)), ((), ())),
                            preferred_element_type=jnp.float32)
        o_ref[...] = y + b_ref[...]

    return pl.pallas_call(
        body,
        grid=(N // blk,),
        in_specs=[
            pl.BlockSpec((blk, D), lambda i: (i, 0)),
            pl.BlockSpec((blk, D), lambda i: (i, 0)),
            pl.BlockSpec((DO, D), lambda i: (0, 0)),
            pl.BlockSpec((1, DO), lambda i: (0, 0)),
        ],
        out_specs=pl.BlockSpec((blk, DO), lambda i: (i, 0)),
        out_shape=jax.ShapeDtypeStruct((N, DO), jnp.float32),
    )(p0, p1, W, b2d)


def kernel(edge_index, adj_values, features, W, b):
    N = features.shape[0]
    partials = _sc_spmm(edge_index[0], edge_index[1], adj_values, features)
    npad = partials.shape[0] // _NC
    return _linear(partials[:N], partials[npad:npad + N], W, b.reshape(1, -1))


# speedup vs baseline: 11.9312x; 1.0669x over previous
import functools

import jax
import jax.numpy as jnp
from jax import lax
from jax.experimental import pallas as pl
from jax.experimental.pallas import tpu as pltpu
from jax.experimental.pallas import tpu_sc as plsc

_NC = 2
_NS = 16
_NW = _NC * _NS
_K = 80


def _sc_spmm(rows, cols, vals, features):
    E = vals.shape[0]
    N, D = features.shape
    assert E % (_NW * _K) == 0
    assert D % 16 == 0
    ew = E // _NW
    nch = ew // _K
    ngroups = (nch - 5) // 12
    rem = nch - ngroups * 12
    assert rem == 5
    rb = 32
    rpt = ((N + _NS * rb - 1) // (_NS * rb)) * rb
    npad = _NS * rpt
    ncp = rpt // rb

    mesh = plsc.VectorSubcoreMesh(core_axis_name="c", subcore_axis_name="s")

    @functools.partial(
        pl.kernel,
        out_type=jax.ShapeDtypeStruct((_NC * npad, D), jnp.float32),
        mesh=mesh,
        compiler_params=pltpu.CompilerParams(needs_layout_passes=False),
        scratch_types=(
            [pltpu.VMEM((_K,), jnp.int32)] * 12
            + [pltpu.VMEM((_K,), jnp.float32)] * 6
            + [
                pltpu.VMEM((4, _K, D), jnp.float32),
                pltpu.VMEM((rb, D), jnp.float32),
                pltpu.VMEM_SHARED((npad, D), jnp.float32),
            ]
            + [pltpu.SemaphoreType.DMA] * 14
        ),
    )
    def spmm(rows_hbm, cols_hbm, vals_hbm, feat_hbm, out_hbm,
             c0, c1, c2, c3, c4, c5, r0, r1, r2, r3, r4, r5,
             v0, v1, v2, v3, v4, v5, gbuf, zbuf, acc,
             gs0, gs1, gs2, gs3, is0, is1, is2, is3, is4, is5,
             ss0, ss1, ss2, ss3):
        cid = lax.axis_index("c")
        sid = lax.axis_index("s")
        wid = sid * _NC + cid

        def zrow(r, carry):
            for cc in range(D // 16):
                zbuf[r, pl.ds(cc * 16, 16)] = jnp.zeros((16,), jnp.float32)
            return carry
        lax.fori_loop(0, rb, zrow, 0)
        for j in range(ncp):
            pltpu.sync_copy(zbuf, acc.at[pl.ds(sid * rpt + j * rb, rb)])
        plsc.subcore_barrier()

        bufs = [gbuf.at[0], gbuf.at[1], gbuf.at[2], gbuf.at[3]]
        cidxs = [c0, c1, c2, c3, c4, c5]
        ridxs = [r0, r1, r2, r3, r4, r5]
        vslots = [v0, v1, v2, v3, v4, v5]
        gsems = [gs0, gs1, gs2, gs3]
        isems = [is0, is1, is2, is3, is4, is5]
        ssems = [ss0, ss1, ss2, ss3]

        def start_idx(ch, s):
            eb = pl.multiple_of(wid * ew + ch * _K, 8)
            pltpu.async_copy(cols_hbm.at[pl.ds(eb, _K)], cidxs[s], isems[s])
            pltpu.async_copy(rows_hbm.at[pl.ds(eb, _K)], ridxs[s], isems[s])
            pltpu.async_copy(vals_hbm.at[pl.ds(eb, _K)], vslots[s], isems[s])

        def wait_idx(ch, s):
            eb = pl.multiple_of(wid * ew + ch * _K, 8)
            pltpu.make_async_copy(cols_hbm.at[pl.ds(eb, _K)], cidxs[s],
                                  isems[s]).wait()
            pltpu.make_async_copy(rows_hbm.at[pl.ds(eb, _K)], ridxs[s],
                                  isems[s]).wait()
            pltpu.make_async_copy(vals_hbm.at[pl.ds(eb, _K)], vslots[s],
                                  isems[s]).wait()

        def start_gather(s, b):
            pltpu.async_copy(feat_hbm.at[cidxs[s]], bufs[b], gsems[b])

        def wait_gather(s, b):
            pltpu.make_async_copy(feat_hbm.at[cidxs[s]], bufs[b],
                                  gsems[b]).wait()

        def scale(s, b):
            buf = bufs[b]
            vv_ref = vslots[s]
            unroll = 4

            def edge_body(i, c2):
                e0 = i * unroll
                vvs = [plsc.load_gather(
                    vv_ref, [jnp.full((16,), e0 + u, jnp.int32)])
                    for u in range(unroll)]
                for cc in range(D // 16):
                    sl = pl.ds(cc * 16, 16)
                    for u in range(unroll):
                        buf[e0 + u, sl] = buf[e0 + u, sl] * vvs[u]
                return c2
            lax.fori_loop(0, _K // unroll, edge_body, 0)

        def start_scatter(s, b):
            pltpu.async_copy(bufs[b], acc.at[ridxs[s]], ssems[b], add=True)

        def wait_scatter(s, b):
            pltpu.make_async_copy(bufs[b], acc.at[ridxs[s]], ssems[b]).wait()

        for s in range(6):
            start_idx(s, s)
        for s in range(3):
            wait_idx(s, s)
            start_gather(s, s)

        def position(ch, p):
            b4 = p % 4
            s6 = p % 6
            wait_gather(s6, b4)
            scale(s6, b4)
            start_scatter(s6, b4)

            @pl.when(ch >= 1)
            def _():
                wait_scatter((p + 5) % 6, (p + 3) % 4)

            @pl.when(jnp.logical_and(ch >= 1, ch + 5 < nch))
            def _():
                start_idx(ch + 5, (p + 5) % 6)

            @pl.when(ch + 3 < nch)
            def _():
                wait_idx(ch + 3, (p + 3) % 6)
                start_gather((p + 3) % 6, (p + 3) % 4)

        def group_body(g, carry):
            ch0 = g * 12
            for p in range(12):
                position(ch0 + p, p)
            return carry
        lax.fori_loop(0, ngroups, group_body, 0)

        for p in range(rem):
            position(ngroups * 12 + p, p)
        wait_scatter((nch - 1) % 6, (nch - 1) % 4)
        plsc.subcore_barrier()

        for j in range(ncp):
            rr = sid * rpt + j * rb
            pltpu.sync_copy(acc.at[pl.ds(rr, rb)], zbuf)
            pltpu.sync_copy(zbuf, out_hbm.at[pl.ds(cid * npad + rr, rb)])

    return spmm(rows, cols, vals, features)


def _linear(p0, p1, W, b2d):
    N, D = p0.shape
    DO = W.shape[0]
    blk = 1000

    def body(p0_ref, p1_ref, w_ref, b_ref, o_ref):
        x = p0_ref[...] + p1_ref[...]
        y = lax.dot_general(x, w_ref[...], (((1,), (1,)), ((), ())),
                            preferred_element_type=jnp.float32)
        o_ref[...] = y + b_ref[...]

    return pl.pallas_call(
        body,
        grid=(N // blk,),
        in_specs=[
            pl.BlockSpec((blk, D), lambda i: (i, 0)),
            pl.BlockSpec((blk, D), lambda i: (i, 0)),
            pl.BlockSpec((DO, D), lambda i: (0, 0)),
            pl.BlockSpec((1, DO), lambda i: (0, 0)),
        ],
        out_specs=pl.BlockSpec((blk, DO), lambda i: (i, 0)),
        out_shape=jax.ShapeDtypeStruct((N, DO), jnp.float32),
    )(p0, p1, W, b2d)


def kernel(edge_index, adj_values, features, W, b):
    N = features.shape[0]
    partials = _sc_spmm(edge_index[0], edge_index[1], adj_values, features)
    npad = partials.shape[0] // _NC
    return _linear(partials[:N], partials[npad:npad + N], W, b.reshape(1, -1))
